# baseline (speedup 1.0000x reference)
import jax
import jax.numpy as jnp
from jax import lax
from jax.experimental import pallas as pl
from jax.experimental.pallas import tpu as pltpu, tpu_sc as plsc

VOCAB = 100000
N_EMBD = 128
B, T = 1024, 200

NC, NS, L = 2, 16, 16
NW = NC * NS
ROWS = B * T
ROWS_W = ROWS // NW
HALF = T // 2
NH = ROWS_W // HALF
NBUF = 4
NV = N_EMBD // L

_EPS = 1e-5
_RSQRT_MAGIC = 0x5F3759DF


def _rsqrt16(x):
    i = plsc.bitcast(x, jnp.int32)
    y = plsc.bitcast(jnp.int32(_RSQRT_MAGIC) - (i >> 1), jnp.float32)
    for _ in range(2):
        y = y * (1.5 - 0.5 * x * y * y)
    return y


def _sc_body(ids_hbm, tt_hbm, tok_hbm, pos_hbm, seg_hbm, gam_hbm, bet_hbm,
             out_hbm, idx_v, ttv_v, bufs, ps_v, seg_v, gb_v, gsems, osems):
    wid = lax.axis_index("s") * NC + lax.axis_index("c")
    obase = wid * NH

    pltpu.sync_copy(ids_hbm.at[pl.ds(wid * NH, NH)], idx_v)
    pltpu.sync_copy(tt_hbm.at[pl.ds(wid * ROWS_W, ROWS_W)],
                    ttv_v.at[pl.ds(0, ROWS_W)])
    pltpu.sync_copy(pos_hbm.at[pl.ds(0, T)], ps_v.at[0])
    pltpu.sync_copy(pos_hbm.at[pl.ds(0, T)], ps_v.at[1])
    pltpu.sync_copy(seg_hbm, seg_v)
    pltpu.sync_copy(gam_hbm, gb_v.at[0])
    pltpu.sync_copy(bet_hbm, gb_v.at[1])

    seg0 = [seg_v[0, pl.ds(L * k, L)] for k in range(NV)]
    seg1 = [seg_v[1, pl.ds(L * k, L)] for k in range(NV)]
    gam = [gb_v[0, pl.ds(L * k, L)] for k in range(NV)]
    bet = [gb_v[1, pl.ds(L * k, L)] for k in range(NV)]

    def build_ps(r, _):
        for k in range(NV):
            ps_v[0, r, pl.ds(L * k, L)] = ps_v[0, r, pl.ds(L * k, L)] + seg0[k]
            ps_v[1, r, pl.ds(L * k, L)] = ps_v[1, r, pl.ds(L * k, L)] + seg1[k]
        return 0

    lax.fori_loop(0, T, build_ps, 0, unroll=2)

    def fire_gather(h, b):
        return pltpu.async_copy(tok_hbm.at[idx_v.at[h]], bufs[b], gsems[b])

    def ln_half(buf, pb, tt0):
        def do_pair(p, _):
            j = 2 * p
            ttpair = ttv_v[pl.ds(tt0 + j, L)]

            def ln_row(j, tts):
                acc = [buf[j, pl.ds(L * k, L)]
                       + ps_v[tts, pb + j, pl.ds(L * k, L)]
                       for k in range(NV)]
                s = ((acc[0] + acc[1]) + (acc[2] + acc[3])) + \
                    ((acc[4] + acc[5]) + (acc[6] + acc[7]))
                sq = None
                for k in range(NV):
                    d2 = acc[k] * acc[k]
                    sq = d2 if sq is None else sq + d2
                mean = jnp.sum(s) * (1.0 / N_EMBD)
                ex2 = jnp.sum(sq) * (1.0 / N_EMBD)
                var = ex2 - mean * mean
                rstd = _rsqrt16(jnp.full((L,), var + _EPS, jnp.float32))
                for k in range(NV):
                    a = rstd * gam[k]
                    buf[j, pl.ds(L * k, L)] = acc[k] * a + (bet[k] - mean * a)

            ln_row(j, ttpair[0])
            ln_row(j + 1, ttpair[1])
            return 0

        lax.fori_loop(0, HALF // 2, do_pair, 0)

    fire_gather(0, 0)
    fire_gather(1, 1)

    def super_step(g, _):
        for i in range(NBUF):
            h = NBUF * g + i
            pltpu.make_async_copy(tok_hbm.at[idx_v.at[0]], bufs[i],
                                  gsems[i]).wait()
            ln_half(bufs[i], (i % 2) * HALF, h * HALF)
            b2 = (i + 2) % NBUF

            @pl.when(h >= 2)
            def _():
                pltpu.make_async_copy(bufs[b2], out_hbm.at[0],
                                      osems[b2]).wait()

            @pl.when(h + 2 < NH)
            def _():
                fire_gather(h + 2, b2)

            pltpu.async_copy(bufs[i], out_hbm.at[obase + h], osems[i])
        return 0

    lax.fori_loop(0, NH // NBUF, super_step, 0)

    pltpu.make_async_copy(bufs[2], out_hbm.at[0], osems[2]).wait()
    pltpu.make_async_copy(bufs[3], out_hbm.at[0], osems[3]).wait()


@jax.jit
def _bert_embed_sc(ids2, tt2, tok_table, pos_table, seg_table, gamma, beta):
    kern = pl.kernel(
        _sc_body,
        out_type=jax.ShapeDtypeStruct((ROWS // HALF, HALF, N_EMBD),
                                      jnp.float32),
        mesh=plsc.VectorSubcoreMesh(core_axis_name="c", subcore_axis_name="s"),
        compiler_params=pltpu.CompilerParams(needs_layout_passes=False),
        scratch_types=[
            pltpu.VMEM((NH, HALF), jnp.int32),
            pltpu.VMEM((ROWS_W + L,), jnp.int32),
            [pltpu.VMEM((HALF, N_EMBD), jnp.float32)
             for _ in range(NBUF)],
            pltpu.VMEM((2, T, N_EMBD), jnp.float32),
            pltpu.VMEM((2, N_EMBD), jnp.float32),
            pltpu.VMEM((2, N_EMBD), jnp.float32),
            [pltpu.SemaphoreType.DMA for _ in range(NBUF)],
            [pltpu.SemaphoreType.DMA for _ in range(NBUF)],
        ],
    )
    return kern(ids2, tt2, tok_table, pos_table, seg_table, gamma, beta)


def kernel(input_ids, token_type_ids, tok_table, pos_table, seg_table,
           ln_gamma, ln_beta):
    ids2 = jnp.asarray(input_ids, jnp.int32).reshape(ROWS // HALF, HALF)
    tt2 = jnp.asarray(token_type_ids, jnp.int32).reshape(ROWS)
    out = _bert_embed_sc(ids2, tt2, tok_table, pos_table, seg_table,
                         jnp.asarray(ln_gamma, jnp.float32),
                         jnp.asarray(ln_beta, jnp.float32))
    return out.reshape(B, T, N_EMBD)


# speedup vs baseline: 1.0879x; 1.0879x over previous
import jax
import jax.numpy as jnp
from jax import lax
from jax.experimental import pallas as pl
from jax.experimental.pallas import tpu as pltpu, tpu_sc as plsc

VOCAB = 100000
N_EMBD = 128
B, T = 1024, 200

NC, NS, L = 2, 16, 16
NW = NC * NS
ROWS = B * T
ROWS_W = ROWS // NW
HALF = T // 2
NH = ROWS_W // HALF
NBUF = 4
NV = N_EMBD // L

_EPS = 1e-5
_RSQRT_MAGIC = 0x5F3759DF


def _rsqrt16(x):
    i = plsc.bitcast(x, jnp.int32)
    y = plsc.bitcast(jnp.int32(_RSQRT_MAGIC) - (i >> 1), jnp.float32)
    for _ in range(2):
        y = y * (1.5 - 0.5 * x * y * y)
    return y


def _sc_body(ids_hbm, tt_hbm, tok_hbm, pos_hbm, seg_hbm, gam_hbm, bet_hbm,
             out_hbm, idx_v, ttv_v, bufs, ps_v, seg_v, gb_v, gsems, osems):
    wid = lax.axis_index("s") * NC + lax.axis_index("c")
    obase = wid * NH

    pltpu.sync_copy(ids_hbm.at[pl.ds(wid * NH, NH)], idx_v)
    pltpu.sync_copy(tt_hbm.at[pl.ds(wid * ROWS_W, ROWS_W)],
                    ttv_v.at[pl.ds(0, ROWS_W)])
    pltpu.sync_copy(pos_hbm.at[pl.ds(0, T)], ps_v.at[0])
    pltpu.sync_copy(pos_hbm.at[pl.ds(0, T)], ps_v.at[1])
    pltpu.sync_copy(seg_hbm, seg_v)
    pltpu.sync_copy(gam_hbm, gb_v.at[0])
    pltpu.sync_copy(bet_hbm, gb_v.at[1])

    lanes = lax.iota(jnp.int32, L)
    perms = [lanes ^ m for m in (8, 4, 2, 1)]

    def lane_sum(v):
        for p in perms:
            v = v + v.at[p].get(mode="promise_in_bounds", unique_indices=True)
        return v

    seg0 = [seg_v[0, pl.ds(L * k, L)] for k in range(NV)]
    seg1 = [seg_v[1, pl.ds(L * k, L)] for k in range(NV)]
    gam = [gb_v[0, pl.ds(L * k, L)] for k in range(NV)]
    bet = [gb_v[1, pl.ds(L * k, L)] for k in range(NV)]

    def build_ps(r, _):
        for k in range(NV):
            ps_v[0, r, pl.ds(L * k, L)] = ps_v[0, r, pl.ds(L * k, L)] + seg0[k]
            ps_v[1, r, pl.ds(L * k, L)] = ps_v[1, r, pl.ds(L * k, L)] + seg1[k]
        return 0

    lax.fori_loop(0, T, build_ps, 0, unroll=2)

    def fire_gather(h, b):
        return pltpu.async_copy(tok_hbm.at[idx_v.at[h]], bufs[b], gsems[b])

    def ln_half(buf, pb, tt0):
        def do_pair(p, _):
            j = 2 * p
            ttpair = ttv_v[pl.ds(tt0 + j, L)]

            def ln_row(j, tts):
                acc = [buf[j, pl.ds(L * k, L)]
                       + ps_v[tts, pb + j, pl.ds(L * k, L)]
                       for k in range(NV)]
                s = ((acc[0] + acc[1]) + (acc[2] + acc[3])) + \
                    ((acc[4] + acc[5]) + (acc[6] + acc[7]))
                sq = None
                for k in range(NV):
                    d2 = acc[k] * acc[k]
                    sq = d2 if sq is None else sq + d2
                mean = lane_sum(s) * (1.0 / N_EMBD)
                ex2 = lane_sum(sq) * (1.0 / N_EMBD)
                var = ex2 - mean * mean
                rstd = _rsqrt16(var + _EPS)
                for k in range(NV):
                    a = rstd * gam[k]
                    buf[j, pl.ds(L * k, L)] = acc[k] * a + (bet[k] - mean * a)

            ln_row(j, ttpair[0])
            ln_row(j + 1, ttpair[1])
            return 0

        lax.fori_loop(0, HALF // 2, do_pair, 0)

    fire_gather(0, 0)
    fire_gather(1, 1)

    def super_step(g, _):
        for i in range(NBUF):
            h = NBUF * g + i
            pltpu.make_async_copy(tok_hbm.at[idx_v.at[0]], bufs[i],
                                  gsems[i]).wait()
            ln_half(bufs[i], (i % 2) * HALF, h * HALF)
            b2 = (i + 2) % NBUF

            @pl.when(h >= 2)
            def _():
                pltpu.make_async_copy(bufs[b2], out_hbm.at[0],
                                      osems[b2]).wait()

            @pl.when(h + 2 < NH)
            def _():
                fire_gather(h + 2, b2)

            pltpu.async_copy(bufs[i], out_hbm.at[obase + h], osems[i])
        return 0

    lax.fori_loop(0, NH // NBUF, super_step, 0)

    pltpu.make_async_copy(bufs[2], out_hbm.at[0], osems[2]).wait()
    pltpu.make_async_copy(bufs[3], out_hbm.at[0], osems[3]).wait()


@jax.jit
def _bert_embed_sc(ids2, tt2, tok_table, pos_table, seg_table, gamma, beta):
    kern = pl.kernel(
        _sc_body,
        out_type=jax.ShapeDtypeStruct((ROWS // HALF, HALF, N_EMBD),
                                      jnp.float32),
        mesh=plsc.VectorSubcoreMesh(core_axis_name="c", subcore_axis_name="s"),
        compiler_params=pltpu.CompilerParams(needs_layout_passes=False),
        scratch_types=[
            pltpu.VMEM((NH, HALF), jnp.int32),
            pltpu.VMEM((ROWS_W + L,), jnp.int32),
            [pltpu.VMEM((HALF, N_EMBD), jnp.float32)
             for _ in range(NBUF)],
            pltpu.VMEM((2, T, N_EMBD), jnp.float32),
            pltpu.VMEM((2, N_EMBD), jnp.float32),
            pltpu.VMEM((2, N_EMBD), jnp.float32),
            [pltpu.SemaphoreType.DMA for _ in range(NBUF)],
            [pltpu.SemaphoreType.DMA for _ in range(NBUF)],
        ],
    )
    return kern(ids2, tt2, tok_table, pos_table, seg_table, gamma, beta)


def kernel(input_ids, token_type_ids, tok_table, pos_table, seg_table,
           ln_gamma, ln_beta):
    ids2 = jnp.asarray(input_ids, jnp.int32).reshape(ROWS // HALF, HALF)
    tt2 = jnp.asarray(token_type_ids, jnp.int32).reshape(ROWS)
    out = _bert_embed_sc(ids2, tt2, tok_table, pos_table, seg_table,
                         jnp.asarray(ln_gamma, jnp.float32),
                         jnp.asarray(ln_beta, jnp.float32))
    return out.reshape(B, T, N_EMBD)
